# baseline (speedup 1.0000x reference)
import functools

import jax
import jax.numpy as jnp
from jax import lax
from jax.experimental import pallas as pl
from jax.experimental.pallas import tpu as pltpu
from jax.experimental.pallas import tpu_sc as plsc

N_NODE = 10000
D_FEAT = 128
D_OUT = 3
D_OUT_PAD = 16

NC, NS = 2, 16
NW = NC * NS
EPW = N_NODE
CH = 128
NCH = EPW // CH
TAIL = EPW - NCH * CH
ROWS_PT = N_NODE // NS
RB = 2048


def _mesh():
  return plsc.VectorSubcoreMesh(core_axis_name="c", subcore_axis_name="s",
                                num_cores=NC, num_subcores=NS)


_SC_PARAMS = pltpu.CompilerParams(use_tc_tiling_on_sc=False,
                                  needs_layout_passes=False)


def _worker_id():
  return lax.axis_index("s") * NC + lax.axis_index("c")


def _zero_rows(buf, nrows, width, dtype):
  lanes = 16 if dtype == jnp.float32 else 32
  zv = jnp.zeros((lanes,), dtype)

  def row(i, _):
    for cc in range(width // lanes):
      buf[i, pl.ds(cc * lanes, lanes)] = zv
    return 0

  lax.fori_loop(0, nrows, row, 0)


def _zero_acc(buf, acc, row0):
  n_full = ROWS_PT // CH
  rem = ROWS_PT - n_full * CH
  for t in range(n_full):
    pltpu.sync_copy(buf, acc.at[pl.ds(row0 + t * CH, CH)])
  pltpu.sync_copy(buf.at[pl.ds(0, rem)], acc.at[pl.ds(row0 + n_full * CH, rem)])


_DEG_GRP = 6


def _deg_body(ei_hbm, ones_hbm, zeros_hbm, out_hbm, didx, ones_v, di_t, acc,
              sem):
  c = lax.axis_index("c")
  s = lax.axis_index("s")
  wid = _worker_id()
  e0 = wid * EPW
  row0 = s * ROWS_PT
  pltpu.sync_copy(ei_hbm.at[1, pl.ds(e0, EPW)], didx)
  pltpu.sync_copy(ei_hbm.at[1, pl.ds(e0 + NCH * CH, TAIL)], di_t)
  pltpu.sync_copy(ones_hbm, ones_v)
  pltpu.sync_copy(zeros_hbm, acc.at[pl.ds(row0, ROWS_PT)])
  plsc.subcore_barrier()

  def group(gi, _):
    j0 = gi * _DEG_GRP
    for r in range(_DEG_GRP):
      pltpu.async_copy(ones_v, acc.at[didx.at[pl.ds((j0 + r) * CH, CH)]], sem,
                       add=True)
    for r in range(_DEG_GRP):
      pltpu.make_async_copy(ones_v, acc.at[didx.at[pl.ds((j0 + r) * CH, CH)]],
                            sem).wait()
    return 0

  lax.fori_loop(0, NCH // _DEG_GRP, group, 0)
  pltpu.sync_copy(ones_v.at[pl.ds(0, TAIL)], acc.at[di_t], add=True)
  plsc.subcore_barrier()
  pltpu.sync_copy(acc.at[pl.ds(row0, ROWS_PT)], out_hbm.at[c, pl.ds(row0, ROWS_PT)])


def _deg_kernel(ei):
  ones = jnp.ones((CH, 8), jnp.float32)
  zeros = jnp.zeros((ROWS_PT, 8), jnp.float32)
  k = functools.partial(
      pl.kernel,
      out_type=jax.ShapeDtypeStruct((NC, N_NODE, 8), jnp.float32),
      mesh=_mesh(),
      compiler_params=_SC_PARAMS,
      scratch_types=[
          pltpu.VMEM((EPW,), jnp.int32),
          pltpu.VMEM((CH, 8), jnp.float32),
          pltpu.VMEM((TAIL,), jnp.int32),
          pltpu.VMEM_SHARED((N_NODE, 8), jnp.float32),
          pltpu.SemaphoreType.DMA,
      ],
  )(_deg_body)
  return k(ei, ones, zeros)


def _make_scatter_body(width, ring, sidx_halves, dtype, stage_table):
  sch = NCH // sidx_halves
  bf16 = dtype == jnp.bfloat16

  def body(table_hbm, ei_hbm, out_hbm, sidx, didx, di_t, *rest):
    rest = list(rest)
    bufs = rest[:ring]
    sems = rest[ring:2 * ring]
    rest = rest[2 * ring:]
    f32buf = rest.pop(0) if bf16 else None
    tspm = rest.pop(0) if stage_table else None
    acc = rest.pop(0)
    c = lax.axis_index("c")
    s = lax.axis_index("s")
    wid = _worker_id()
    e0 = wid * EPW
    row0 = s * ROWS_PT

    _zero_rows(bufs[0], CH, width, dtype)
    _zero_acc(bufs[0], acc, row0)
    pltpu.sync_copy(ei_hbm.at[1, pl.ds(e0, EPW)], didx)
    pltpu.sync_copy(ei_hbm.at[1, pl.ds(e0 + NCH * CH, TAIL)], di_t)
    if stage_table:
      pltpu.sync_copy(table_hbm.at[pl.ds(row0, ROWS_PT)],
                      tspm.at[pl.ds(row0, ROWS_PT)])
    plsc.subcore_barrier()
    table = tspm if stage_table else table_hbm

    for h in range(sidx_halves):
      n_src = sch * CH + (TAIL if h == sidx_halves - 1 else 0)
      pltpu.sync_copy(ei_hbm.at[0, pl.ds(e0 + h * sch * CH, n_src)],
                      sidx.at[pl.ds(0, n_src)])

      for r in range(ring):
        pltpu.async_copy(table.at[sidx.at[pl.ds(r * CH, CH)]], bufs[r],
                         sems[r])

      def outer(jo, _):
        j0 = jo * ring
        for r in range(ring):
          j = j0 + r
          pltpu.make_async_copy(table.at[sidx.at[pl.ds(j * CH, CH)]],
                                bufs[r], sems[r]).wait()
          pltpu.sync_copy(bufs[r],
                          acc.at[didx.at[pl.ds((h * sch + j) * CH, CH)]],
                          add=True)

          @pl.when(j + ring < sch)
          def _():
            pltpu.async_copy(table.at[sidx.at[pl.ds((j + ring) * CH, CH)]],
                             bufs[r], sems[r])

        return 0

      lax.fori_loop(0, sch // ring, outer, 0)

    pltpu.async_copy(table.at[sidx.at[pl.ds(sch * CH, TAIL)]],
                     bufs[0].at[pl.ds(0, TAIL)], sems[0]).wait()
    pltpu.sync_copy(bufs[0].at[pl.ds(0, TAIL)], acc.at[di_t], add=True)

    plsc.subcore_barrier()
    if not bf16:
      pltpu.sync_copy(acc.at[pl.ds(row0, ROWS_PT)],
                      out_hbm.at[c, pl.ds(row0, ROWS_PT)])
      return

    iota = lax.iota(jnp.int32, 16)
    hi_mask = jnp.full((16,), -65536, jnp.int32)
    sixteen = jnp.full((16,), 16, jnp.int32)

    def convert_span(r0, nrows):
      pltpu.sync_copy(acc.at[pl.ds(r0, nrows)], bufs[0].at[pl.ds(0, nrows)])

      def row(i, _):
        rowv = jnp.full((16,), i, jnp.int32)
        for gk in range(width // 32):
          v = bufs[0][i, pl.ds(gk * 32, 32)]
          w = plsc.bitcast(v, jnp.int32)
          ev = plsc.bitcast(lax.shift_left(w, sixteen), jnp.float32)
          od = plsc.bitcast(lax.bitwise_and(w, hi_mask), jnp.float32)
          cols = gk * 32 + 2 * iota
          plsc.store_scatter(f32buf, [rowv, cols], ev)
          plsc.store_scatter(f32buf, [rowv, cols + 1], od)
        return 0

      lax.fori_loop(0, nrows, row, 0)
      pltpu.sync_copy(f32buf.at[pl.ds(0, nrows)],
                      out_hbm.at[c, pl.ds(r0, nrows)])

    n_full = ROWS_PT // CH
    for t in range(n_full):
      convert_span(row0 + t * CH, CH)
    convert_span(row0 + n_full * CH, ROWS_PT - n_full * CH)

  return body


def _scatter_kernel(table, ei, width, ring, sidx_halves, dtype=jnp.float32,
                    stage_table=False):
  sidx_len = (NCH // sidx_halves) * CH + TAIL
  f32buf = [pltpu.VMEM((CH, width), jnp.float32)] if dtype == jnp.bfloat16 else []
  tspm = [pltpu.VMEM_SHARED((N_NODE, width), dtype)] if stage_table else []
  k = functools.partial(
      pl.kernel,
      out_type=jax.ShapeDtypeStruct((NC, N_NODE, width), jnp.float32),
      mesh=_mesh(),
      compiler_params=_SC_PARAMS,
      scratch_types=[
          pltpu.VMEM((sidx_len,), jnp.int32),
          pltpu.VMEM((EPW,), jnp.int32),
          pltpu.VMEM((TAIL,), jnp.int32),
          *[pltpu.VMEM((CH, width), dtype) for _ in range(ring)],
          *[pltpu.SemaphoreType.DMA for _ in range(ring)],
          *f32buf,
          *tspm,
          pltpu.VMEM_SHARED((N_NODE, width), dtype),
      ],
  )(_make_scatter_body(width, ring, sidx_halves, dtype, stage_table))
  return k(table, ei)


_GRID = (N_NODE + RB - 1) // RB


def _tc1_body(x_ref, w_ref, degp_ref, g_ref, d_ref):
  d = lax.rsqrt(degp_ref[0, :, 0] + degp_ref[1, :, 0] + 1.0)
  h = jnp.dot(x_ref[...], w_ref[...], preferred_element_type=jnp.float32)
  g_ref[...] = (h * d[:, None]).astype(jnp.bfloat16)
  d_ref[...] = d


def _tc1(x, W1, degp):
  return pl.pallas_call(
      _tc1_body,
      grid=(_GRID,),
      in_specs=[
          pl.BlockSpec((RB, D_FEAT), lambda i: (i, 0)),
          pl.BlockSpec((D_FEAT, D_FEAT), lambda i: (0, 0)),
          pl.BlockSpec((NC, RB, 8), lambda i: (0, i, 0)),
      ],
      out_specs=[
          pl.BlockSpec((RB, D_FEAT), lambda i: (i, 0)),
          pl.BlockSpec((RB,), lambda i: (i,)),
      ],
      out_shape=[
          jax.ShapeDtypeStruct((N_NODE, D_FEAT), jnp.bfloat16),
          jax.ShapeDtypeStruct((N_NODE,), jnp.float32),
      ],
  )(x, W1, degp)


def _tc2_body(g_ref, sp_ref, d_ref, b1_ref, w2_ref, g2_ref):
  d = d_ref[...]
  msum = sp_ref[0] + sp_ref[1] + g_ref[...].astype(jnp.float32)
  z = d[:, None] * msum + b1_ref[...][None, :]
  z = jnp.maximum(z, 0.0)
  h2 = jnp.dot(z, w2_ref[...], preferred_element_type=jnp.float32)
  g2_ref[...] = h2 * d[:, None]


def _tc2(g, sp, d, b1, W2p):
  return pl.pallas_call(
      _tc2_body,
      grid=(_GRID,),
      in_specs=[
          pl.BlockSpec((RB, D_FEAT), lambda i: (i, 0)),
          pl.BlockSpec((NC, RB, D_FEAT), lambda i: (0, i, 0)),
          pl.BlockSpec((RB,), lambda i: (i,)),
          pl.BlockSpec((D_FEAT,), lambda i: (0,)),
          pl.BlockSpec((D_FEAT, D_OUT_PAD), lambda i: (0, 0)),
      ],
      out_specs=pl.BlockSpec((RB, D_OUT_PAD), lambda i: (i, 0)),
      out_shape=jax.ShapeDtypeStruct((N_NODE, D_OUT_PAD), jnp.float32),
  )(g, sp, d, b1, W2p)


def _tc3_body(g2_ref, s2p_ref, d_ref, b2_ref, y_ref):
  d = d_ref[...]
  y = d[:, None] * (s2p_ref[0] + s2p_ref[1] + g2_ref[...]) + b2_ref[...][None, :]
  y_ref[...] = jnp.tanh(jnp.maximum(y, 0.0))[:, :D_OUT]


def _tc3(g2, s2p, d, b2p):
  return pl.pallas_call(
      _tc3_body,
      grid=(_GRID,),
      in_specs=[
          pl.BlockSpec((RB, D_OUT_PAD), lambda i: (i, 0)),
          pl.BlockSpec((NC, RB, D_OUT_PAD), lambda i: (0, i, 0)),
          pl.BlockSpec((RB,), lambda i: (i,)),
          pl.BlockSpec((D_OUT_PAD,), lambda i: (0,)),
      ],
      out_specs=pl.BlockSpec((RB, D_OUT), lambda i: (i, 0)),
      out_shape=jax.ShapeDtypeStruct((N_NODE, D_OUT), jnp.float32),
  )(g2, s2p, d, b2p)


def kernel(x, edge_index, W1, b1, W2, b2):
  ei = edge_index.astype(jnp.int32)
  W2p = jnp.pad(W2, ((0, 0), (0, D_OUT_PAD - W2.shape[1])))
  b2p = jnp.pad(b2, (0, D_OUT_PAD - b2.shape[0]))

  degp = _deg_kernel(ei)
  g, d = _tc1(x, W1, degp)
  sp = _scatter_kernel(g, ei, D_FEAT, ring=6, sidx_halves=1, dtype=jnp.bfloat16)
  g2 = _tc2(g, sp, d, b1, W2p)
  s2p = _scatter_kernel(g2, ei, D_OUT_PAD, ring=6, sidx_halves=1,
                        stage_table=True)
  y = _tc3(g2, s2p, d, b2p)
  return y


# speedup vs baseline: 1.0116x; 1.0116x over previous
import functools

import jax
import jax.numpy as jnp
from jax import lax
from jax.experimental import pallas as pl
from jax.experimental.pallas import tpu as pltpu
from jax.experimental.pallas import tpu_sc as plsc

N_NODE = 10000
D_FEAT = 128
D_OUT = 3
D_OUT_PAD = 16

NC, NS = 2, 16
NW = NC * NS
EPW = N_NODE
CH = 128
NCH = EPW // CH
TAIL = EPW - NCH * CH
ROWS_PT = N_NODE // NS
RB = 2048


def _mesh():
  return plsc.VectorSubcoreMesh(core_axis_name="c", subcore_axis_name="s",
                                num_cores=NC, num_subcores=NS)


_SC_PARAMS = pltpu.CompilerParams(use_tc_tiling_on_sc=False,
                                  needs_layout_passes=False)


def _worker_id():
  return lax.axis_index("s") * NC + lax.axis_index("c")


def _zero_rows(buf, nrows, width, dtype):
  lanes = 16 if dtype == jnp.float32 else 32
  zv = jnp.zeros((lanes,), dtype)

  def row(i, _):
    for cc in range(width // lanes):
      buf[i, pl.ds(cc * lanes, lanes)] = zv
    return 0

  lax.fori_loop(0, nrows, row, 0)


def _zero_acc(buf, acc, row0):
  n_full = ROWS_PT // CH
  rem = ROWS_PT - n_full * CH
  for t in range(n_full):
    pltpu.sync_copy(buf, acc.at[pl.ds(row0 + t * CH, CH)])
  pltpu.sync_copy(buf.at[pl.ds(0, rem)], acc.at[pl.ds(row0 + n_full * CH, rem)])


_DEG_GRP = 6


def _deg_body(ei_hbm, ones_hbm, zeros_hbm, out_hbm, didx, ones_v, di_t, acc,
              sem):
  c = lax.axis_index("c")
  s = lax.axis_index("s")
  wid = _worker_id()
  e0 = wid * EPW
  row0 = s * ROWS_PT
  pltpu.sync_copy(ei_hbm.at[1, pl.ds(e0, EPW)], didx)
  pltpu.sync_copy(ei_hbm.at[1, pl.ds(e0 + NCH * CH, TAIL)], di_t)
  pltpu.sync_copy(ones_hbm, ones_v)
  pltpu.sync_copy(zeros_hbm, acc.at[pl.ds(row0, ROWS_PT)])
  plsc.subcore_barrier()

  def group(gi, _):
    j0 = gi * _DEG_GRP
    for r in range(_DEG_GRP):
      pltpu.async_copy(ones_v, acc.at[didx.at[pl.ds((j0 + r) * CH, CH)]], sem,
                       add=True)
    for r in range(_DEG_GRP):
      pltpu.make_async_copy(ones_v, acc.at[didx.at[pl.ds((j0 + r) * CH, CH)]],
                            sem).wait()
    return 0

  lax.fori_loop(0, NCH // _DEG_GRP, group, 0)
  pltpu.sync_copy(ones_v.at[pl.ds(0, TAIL)], acc.at[di_t], add=True)
  plsc.subcore_barrier()
  pltpu.sync_copy(acc.at[pl.ds(row0, ROWS_PT)], out_hbm.at[c, pl.ds(row0, ROWS_PT)])


def _deg_kernel(ei):
  ones = jnp.ones((CH, 8), jnp.float32)
  zeros = jnp.zeros((ROWS_PT, 8), jnp.float32)
  k = functools.partial(
      pl.kernel,
      out_type=jax.ShapeDtypeStruct((NC, N_NODE, 8), jnp.float32),
      mesh=_mesh(),
      compiler_params=_SC_PARAMS,
      scratch_types=[
          pltpu.VMEM((EPW,), jnp.int32),
          pltpu.VMEM((CH, 8), jnp.float32),
          pltpu.VMEM((TAIL,), jnp.int32),
          pltpu.VMEM_SHARED((N_NODE, 8), jnp.float32),
          pltpu.SemaphoreType.DMA,
      ],
  )(_deg_body)
  return k(ei, ones, zeros)


def _make_scatter_body(width, ring, sidx_halves, dtype, stage_table):
  sch = NCH // sidx_halves
  bf16 = dtype == jnp.bfloat16

  def body(table_hbm, ei_hbm, out_hbm, sidx, didx, di_t, *rest):
    rest = list(rest)
    bufs = rest[:ring]
    sems = rest[ring:2 * ring]
    rest = rest[2 * ring:]
    f32buf = rest.pop(0) if bf16 else None
    tspm = rest.pop(0) if stage_table else None
    acc = rest.pop(0)
    c = lax.axis_index("c")
    s = lax.axis_index("s")
    wid = _worker_id()
    e0 = wid * EPW
    row0 = s * ROWS_PT

    _zero_rows(bufs[0], CH, width, dtype)
    _zero_acc(bufs[0], acc, row0)
    pltpu.sync_copy(ei_hbm.at[1, pl.ds(e0, EPW)], didx)
    pltpu.sync_copy(ei_hbm.at[1, pl.ds(e0 + NCH * CH, TAIL)], di_t)
    if stage_table:
      pltpu.sync_copy(table_hbm.at[pl.ds(row0, ROWS_PT)],
                      tspm.at[pl.ds(row0, ROWS_PT)])
    plsc.subcore_barrier()
    table = tspm if stage_table else table_hbm

    for h in range(sidx_halves):
      n_src = sch * CH + (TAIL if h == sidx_halves - 1 else 0)
      pltpu.sync_copy(ei_hbm.at[0, pl.ds(e0 + h * sch * CH, n_src)],
                      sidx.at[pl.ds(0, n_src)])

      for r in range(ring):
        pltpu.async_copy(table.at[sidx.at[pl.ds(r * CH, CH)]], bufs[r],
                         sems[r])

      def outer(jo, _):
        j0 = jo * ring
        for r in range(ring):
          j = j0 + r
          pltpu.make_async_copy(table.at[sidx.at[pl.ds(j * CH, CH)]],
                                bufs[r], sems[r]).wait()
          pltpu.sync_copy(bufs[r],
                          acc.at[didx.at[pl.ds((h * sch + j) * CH, CH)]],
                          add=True)

          @pl.when(j + ring < sch)
          def _():
            pltpu.async_copy(table.at[sidx.at[pl.ds((j + ring) * CH, CH)]],
                             bufs[r], sems[r])

        return 0

      lax.fori_loop(0, sch // ring, outer, 0)

    pltpu.async_copy(table.at[sidx.at[pl.ds(sch * CH, TAIL)]],
                     bufs[0].at[pl.ds(0, TAIL)], sems[0]).wait()
    pltpu.sync_copy(bufs[0].at[pl.ds(0, TAIL)], acc.at[di_t], add=True)

    plsc.subcore_barrier()
    if not bf16:
      pltpu.sync_copy(acc.at[pl.ds(row0, ROWS_PT)],
                      out_hbm.at[c, pl.ds(row0, ROWS_PT)])
      return

    iota = lax.iota(jnp.int32, 16)
    hi_mask = jnp.full((16,), -65536, jnp.int32)
    sixteen = jnp.full((16,), 16, jnp.int32)

    def convert_span(r0, nrows):
      pltpu.sync_copy(acc.at[pl.ds(r0, nrows)], bufs[0].at[pl.ds(0, nrows)])

      def row(i, _):
        rowv = jnp.full((16,), i, jnp.int32)
        for gk in range(width // 32):
          v = bufs[0][i, pl.ds(gk * 32, 32)]
          w = plsc.bitcast(v, jnp.int32)
          ev = plsc.bitcast(lax.shift_left(w, sixteen), jnp.float32)
          od = plsc.bitcast(lax.bitwise_and(w, hi_mask), jnp.float32)
          cols = gk * 32 + 2 * iota
          plsc.store_scatter(f32buf, [rowv, cols], ev)
          plsc.store_scatter(f32buf, [rowv, cols + 1], od)
        return 0

      lax.fori_loop(0, nrows, row, 0)
      pltpu.sync_copy(f32buf.at[pl.ds(0, nrows)],
                      out_hbm.at[c, pl.ds(r0, nrows)])

    n_full = ROWS_PT // CH
    for t in range(n_full):
      convert_span(row0 + t * CH, CH)
    convert_span(row0 + n_full * CH, ROWS_PT - n_full * CH)

  return body


def _scatter_kernel(table, ei, width, ring, sidx_halves, dtype=jnp.float32,
                    stage_table=False):
  sidx_len = (NCH // sidx_halves) * CH + TAIL
  f32buf = [pltpu.VMEM((CH, width), jnp.float32)] if dtype == jnp.bfloat16 else []
  tspm = [pltpu.VMEM_SHARED((N_NODE, width), dtype)] if stage_table else []
  k = functools.partial(
      pl.kernel,
      out_type=jax.ShapeDtypeStruct((NC, N_NODE, width), jnp.float32),
      mesh=_mesh(),
      compiler_params=_SC_PARAMS,
      scratch_types=[
          pltpu.VMEM((sidx_len,), jnp.int32),
          pltpu.VMEM((EPW,), jnp.int32),
          pltpu.VMEM((TAIL,), jnp.int32),
          *[pltpu.VMEM((CH, width), dtype) for _ in range(ring)],
          *[pltpu.SemaphoreType.DMA for _ in range(ring)],
          *f32buf,
          *tspm,
          pltpu.VMEM_SHARED((N_NODE, width), dtype),
      ],
  )(_make_scatter_body(width, ring, sidx_halves, dtype, stage_table))
  return k(table, ei)


_GRID = (N_NODE + RB - 1) // RB


def _tc1_body(x_ref, w_ref, degp_ref, g_ref, d_ref):
  d = lax.rsqrt(degp_ref[0, :, 0] + degp_ref[1, :, 0] + 1.0)
  h = jnp.dot(x_ref[...], w_ref[...], preferred_element_type=jnp.float32)
  g_ref[...] = (h * d[:, None]).astype(jnp.bfloat16)
  d_ref[...] = d


def _tc1(x, W1, degp):
  return pl.pallas_call(
      _tc1_body,
      grid=(_GRID,),
      in_specs=[
          pl.BlockSpec((RB, D_FEAT), lambda i: (i, 0)),
          pl.BlockSpec((D_FEAT, D_FEAT), lambda i: (0, 0)),
          pl.BlockSpec((NC, RB, 8), lambda i: (0, i, 0)),
      ],
      out_specs=[
          pl.BlockSpec((RB, D_FEAT), lambda i: (i, 0)),
          pl.BlockSpec((RB,), lambda i: (i,)),
      ],
      out_shape=[
          jax.ShapeDtypeStruct((N_NODE, D_FEAT), jnp.bfloat16),
          jax.ShapeDtypeStruct((N_NODE,), jnp.float32),
      ],
  )(x, W1, degp)


def _tc2_body(g_ref, sp_ref, d_ref, b1_ref, w2_ref, g2_ref):
  d = d_ref[...]
  msum = sp_ref[0] + sp_ref[1] + g_ref[...].astype(jnp.float32)
  z = d[:, None] * msum + b1_ref[...][None, :]
  z = jnp.maximum(z, 0.0)
  h2 = jnp.dot(z, w2_ref[...], preferred_element_type=jnp.float32)
  g2_ref[...] = h2 * d[:, None]


def _tc2(g, sp, d, b1, W2p):
  return pl.pallas_call(
      _tc2_body,
      grid=(_GRID,),
      in_specs=[
          pl.BlockSpec((RB, D_FEAT), lambda i: (i, 0)),
          pl.BlockSpec((NC, RB, D_FEAT), lambda i: (0, i, 0)),
          pl.BlockSpec((RB,), lambda i: (i,)),
          pl.BlockSpec((D_FEAT,), lambda i: (0,)),
          pl.BlockSpec((D_FEAT, D_OUT_PAD), lambda i: (0, 0)),
      ],
      out_specs=pl.BlockSpec((RB, D_OUT_PAD), lambda i: (i, 0)),
      out_shape=jax.ShapeDtypeStruct((N_NODE, D_OUT_PAD), jnp.float32),
  )(g, sp, d, b1, W2p)


def _tc3_body(g2_ref, s2p_ref, d_ref, b2_ref, y_ref):
  d = d_ref[...]
  y = d[:, None] * (s2p_ref[0] + s2p_ref[1] + g2_ref[...]) + b2_ref[...][None, :]
  y_ref[...] = jnp.tanh(jnp.maximum(y, 0.0))[:, :D_OUT]


def _tc3(g2, s2p, d, b2p):
  return pl.pallas_call(
      _tc3_body,
      grid=(_GRID,),
      in_specs=[
          pl.BlockSpec((RB, D_OUT_PAD), lambda i: (i, 0)),
          pl.BlockSpec((NC, RB, D_OUT_PAD), lambda i: (0, i, 0)),
          pl.BlockSpec((RB,), lambda i: (i,)),
          pl.BlockSpec((D_OUT_PAD,), lambda i: (0,)),
      ],
      out_specs=pl.BlockSpec((RB, D_OUT), lambda i: (i, 0)),
      out_shape=jax.ShapeDtypeStruct((N_NODE, D_OUT), jnp.float32),
  )(g2, s2p, d, b2p)


def kernel(x, edge_index, W1, b1, W2, b2):
  ei = edge_index.astype(jnp.int32)
  W2p = jnp.pad(W2, ((0, 0), (0, D_OUT_PAD - W2.shape[1])))
  b2p = jnp.pad(b2, (0, D_OUT_PAD - b2.shape[0]))

  degp = _deg_kernel(ei)
  g, d = _tc1(x, W1, degp)
  sp = _scatter_kernel(g, ei, D_FEAT, ring=6, sidx_halves=1, dtype=jnp.bfloat16)
  g2 = _tc2(g, sp, d, b1, W2p)
  s2p = _scatter_kernel(g2, ei, D_OUT_PAD, ring=6, sidx_halves=1)
  y = _tc3(g2, s2p, d, b2p)
  return y


# speedup vs baseline: 1.0254x; 1.0137x over previous
import functools

import jax
import jax.numpy as jnp
from jax import lax
from jax.experimental import pallas as pl
from jax.experimental.pallas import tpu as pltpu
from jax.experimental.pallas import tpu_sc as plsc

N_NODE = 10000
D_FEAT = 128
D_OUT = 3
D_OUT_PAD = 16

NC, NS = 2, 16
NW = NC * NS
EPW = N_NODE
CH = 128
NCH = EPW // CH
TAIL = EPW - NCH * CH
ROWS_PT = N_NODE // NS
RB = 4096


def _mesh():
  return plsc.VectorSubcoreMesh(core_axis_name="c", subcore_axis_name="s",
                                num_cores=NC, num_subcores=NS)


_SC_PARAMS = pltpu.CompilerParams(use_tc_tiling_on_sc=False,
                                  needs_layout_passes=False)


def _worker_id():
  return lax.axis_index("s") * NC + lax.axis_index("c")


def _zero_rows(buf, nrows, width, dtype):
  lanes = 16 if dtype == jnp.float32 else 32
  zv = jnp.zeros((lanes,), dtype)

  def row(i, _):
    for cc in range(width // lanes):
      buf[i, pl.ds(cc * lanes, lanes)] = zv
    return 0

  lax.fori_loop(0, nrows, row, 0)


def _zero_acc(buf, acc, row0):
  n_full = ROWS_PT // CH
  rem = ROWS_PT - n_full * CH
  for t in range(n_full):
    pltpu.sync_copy(buf, acc.at[pl.ds(row0 + t * CH, CH)])
  pltpu.sync_copy(buf.at[pl.ds(0, rem)], acc.at[pl.ds(row0 + n_full * CH, rem)])


_DEG_GRP = 6


def _deg_body(ei_hbm, ones_hbm, zeros_hbm, out_hbm, didx, ones_v, di_t, acc,
              sem):
  c = lax.axis_index("c")
  s = lax.axis_index("s")
  wid = _worker_id()
  e0 = wid * EPW
  row0 = s * ROWS_PT
  pltpu.sync_copy(ei_hbm.at[1, pl.ds(e0, EPW)], didx)
  pltpu.sync_copy(ei_hbm.at[1, pl.ds(e0 + NCH * CH, TAIL)], di_t)
  pltpu.sync_copy(ones_hbm, ones_v)
  pltpu.sync_copy(zeros_hbm, acc.at[pl.ds(row0, ROWS_PT)])
  plsc.subcore_barrier()

  def group(gi, _):
    j0 = gi * _DEG_GRP
    for r in range(_DEG_GRP):
      pltpu.async_copy(ones_v, acc.at[didx.at[pl.ds((j0 + r) * CH, CH)]], sem,
                       add=True)
    for r in range(_DEG_GRP):
      pltpu.make_async_copy(ones_v, acc.at[didx.at[pl.ds((j0 + r) * CH, CH)]],
                            sem).wait()
    return 0

  lax.fori_loop(0, NCH // _DEG_GRP, group, 0)
  pltpu.sync_copy(ones_v.at[pl.ds(0, TAIL)], acc.at[di_t], add=True)
  plsc.subcore_barrier()
  pltpu.sync_copy(acc.at[pl.ds(row0, ROWS_PT)], out_hbm.at[c, pl.ds(row0, ROWS_PT)])


def _deg_kernel(ei):
  ones = jnp.ones((CH, 8), jnp.float32)
  zeros = jnp.zeros((ROWS_PT, 8), jnp.float32)
  k = functools.partial(
      pl.kernel,
      out_type=jax.ShapeDtypeStruct((NC, N_NODE, 8), jnp.float32),
      mesh=_mesh(),
      compiler_params=_SC_PARAMS,
      scratch_types=[
          pltpu.VMEM((EPW,), jnp.int32),
          pltpu.VMEM((CH, 8), jnp.float32),
          pltpu.VMEM((TAIL,), jnp.int32),
          pltpu.VMEM_SHARED((N_NODE, 8), jnp.float32),
          pltpu.SemaphoreType.DMA,
      ],
  )(_deg_body)
  return k(ei, ones, zeros)


def _make_scatter_body(width, ring, sidx_halves, dtype, stage_table):
  sch = NCH // sidx_halves
  bf16 = dtype == jnp.bfloat16

  def body(table_hbm, ei_hbm, out_hbm, sidx, didx, di_t, *rest):
    rest = list(rest)
    bufs = rest[:ring]
    sems = rest[ring:2 * ring]
    rest = rest[2 * ring:]
    f32buf = rest.pop(0) if bf16 else None
    tspm = rest.pop(0) if stage_table else None
    acc = rest.pop(0)
    c = lax.axis_index("c")
    s = lax.axis_index("s")
    wid = _worker_id()
    e0 = wid * EPW
    row0 = s * ROWS_PT

    _zero_rows(bufs[0], CH, width, dtype)
    _zero_acc(bufs[0], acc, row0)
    pltpu.sync_copy(ei_hbm.at[1, pl.ds(e0, EPW)], didx)
    pltpu.sync_copy(ei_hbm.at[1, pl.ds(e0 + NCH * CH, TAIL)], di_t)
    if stage_table:
      pltpu.sync_copy(table_hbm.at[pl.ds(row0, ROWS_PT)],
                      tspm.at[pl.ds(row0, ROWS_PT)])
    plsc.subcore_barrier()
    table = tspm if stage_table else table_hbm

    for h in range(sidx_halves):
      n_src = sch * CH + (TAIL if h == sidx_halves - 1 else 0)
      pltpu.sync_copy(ei_hbm.at[0, pl.ds(e0 + h * sch * CH, n_src)],
                      sidx.at[pl.ds(0, n_src)])

      for r in range(ring):
        pltpu.async_copy(table.at[sidx.at[pl.ds(r * CH, CH)]], bufs[r],
                         sems[r])

      def outer(jo, _):
        j0 = jo * ring
        for r in range(ring):
          j = j0 + r
          pltpu.make_async_copy(table.at[sidx.at[pl.ds(j * CH, CH)]],
                                bufs[r], sems[r]).wait()
          pltpu.sync_copy(bufs[r],
                          acc.at[didx.at[pl.ds((h * sch + j) * CH, CH)]],
                          add=True)

          @pl.when(j + ring < sch)
          def _():
            pltpu.async_copy(table.at[sidx.at[pl.ds((j + ring) * CH, CH)]],
                             bufs[r], sems[r])

        return 0

      lax.fori_loop(0, sch // ring, outer, 0)

    pltpu.async_copy(table.at[sidx.at[pl.ds(sch * CH, TAIL)]],
                     bufs[0].at[pl.ds(0, TAIL)], sems[0]).wait()
    pltpu.sync_copy(bufs[0].at[pl.ds(0, TAIL)], acc.at[di_t], add=True)

    plsc.subcore_barrier()
    if not bf16:
      pltpu.sync_copy(acc.at[pl.ds(row0, ROWS_PT)],
                      out_hbm.at[c, pl.ds(row0, ROWS_PT)])
      return

    iota = lax.iota(jnp.int32, 16)
    hi_mask = jnp.full((16,), -65536, jnp.int32)
    sixteen = jnp.full((16,), 16, jnp.int32)

    def convert_span(r0, nrows):
      pltpu.sync_copy(acc.at[pl.ds(r0, nrows)], bufs[0].at[pl.ds(0, nrows)])

      def row(i, _):
        rowv = jnp.full((16,), i, jnp.int32)
        for gk in range(width // 32):
          v = bufs[0][i, pl.ds(gk * 32, 32)]
          w = plsc.bitcast(v, jnp.int32)
          ev = plsc.bitcast(lax.shift_left(w, sixteen), jnp.float32)
          od = plsc.bitcast(lax.bitwise_and(w, hi_mask), jnp.float32)
          cols = gk * 32 + 2 * iota
          plsc.store_scatter(f32buf, [rowv, cols], ev)
          plsc.store_scatter(f32buf, [rowv, cols + 1], od)
        return 0

      lax.fori_loop(0, nrows, row, 0)
      pltpu.sync_copy(f32buf.at[pl.ds(0, nrows)],
                      out_hbm.at[c, pl.ds(r0, nrows)])

    n_full = ROWS_PT // CH
    for t in range(n_full):
      convert_span(row0 + t * CH, CH)
    convert_span(row0 + n_full * CH, ROWS_PT - n_full * CH)

  return body


def _scatter_kernel(table, ei, width, ring, sidx_halves, dtype=jnp.float32,
                    stage_table=False):
  sidx_len = (NCH // sidx_halves) * CH + TAIL
  f32buf = [pltpu.VMEM((CH, width), jnp.float32)] if dtype == jnp.bfloat16 else []
  tspm = [pltpu.VMEM_SHARED((N_NODE, width), dtype)] if stage_table else []
  k = functools.partial(
      pl.kernel,
      out_type=jax.ShapeDtypeStruct((NC, N_NODE, width), jnp.float32),
      mesh=_mesh(),
      compiler_params=_SC_PARAMS,
      scratch_types=[
          pltpu.VMEM((sidx_len,), jnp.int32),
          pltpu.VMEM((EPW,), jnp.int32),
          pltpu.VMEM((TAIL,), jnp.int32),
          *[pltpu.VMEM((CH, width), dtype) for _ in range(ring)],
          *[pltpu.SemaphoreType.DMA for _ in range(ring)],
          *f32buf,
          *tspm,
          pltpu.VMEM_SHARED((N_NODE, width), dtype),
      ],
  )(_make_scatter_body(width, ring, sidx_halves, dtype, stage_table))
  return k(table, ei)


_GRID = (N_NODE + RB - 1) // RB


def _tc1_body(x_ref, w_ref, degp_ref, g_ref, d_ref):
  d = lax.rsqrt(degp_ref[0, :, 0] + degp_ref[1, :, 0] + 1.0)
  h = jnp.dot(x_ref[...], w_ref[...], preferred_element_type=jnp.float32)
  g_ref[...] = (h * d[:, None]).astype(jnp.bfloat16)
  d_ref[...] = d


def _tc1(x, W1, degp):
  return pl.pallas_call(
      _tc1_body,
      grid=(_GRID,),
      in_specs=[
          pl.BlockSpec((RB, D_FEAT), lambda i: (i, 0)),
          pl.BlockSpec((D_FEAT, D_FEAT), lambda i: (0, 0)),
          pl.BlockSpec((NC, RB, 8), lambda i: (0, i, 0)),
      ],
      out_specs=[
          pl.BlockSpec((RB, D_FEAT), lambda i: (i, 0)),
          pl.BlockSpec((RB,), lambda i: (i,)),
      ],
      out_shape=[
          jax.ShapeDtypeStruct((N_NODE, D_FEAT), jnp.bfloat16),
          jax.ShapeDtypeStruct((N_NODE,), jnp.float32),
      ],
  )(x, W1, degp)


def _tc2_body(g_ref, sp_ref, d_ref, b1_ref, w2_ref, g2_ref):
  d = d_ref[...]
  msum = sp_ref[0] + sp_ref[1] + g_ref[...].astype(jnp.float32)
  z = d[:, None] * msum + b1_ref[...][None, :]
  z = jnp.maximum(z, 0.0)
  h2 = jnp.dot(z, w2_ref[...], preferred_element_type=jnp.float32)
  g2_ref[...] = h2 * d[:, None]


def _tc2(g, sp, d, b1, W2p):
  return pl.pallas_call(
      _tc2_body,
      grid=(_GRID,),
      in_specs=[
          pl.BlockSpec((RB, D_FEAT), lambda i: (i, 0)),
          pl.BlockSpec((NC, RB, D_FEAT), lambda i: (0, i, 0)),
          pl.BlockSpec((RB,), lambda i: (i,)),
          pl.BlockSpec((D_FEAT,), lambda i: (0,)),
          pl.BlockSpec((D_FEAT, D_OUT_PAD), lambda i: (0, 0)),
      ],
      out_specs=pl.BlockSpec((RB, D_OUT_PAD), lambda i: (i, 0)),
      out_shape=jax.ShapeDtypeStruct((N_NODE, D_OUT_PAD), jnp.float32),
  )(g, sp, d, b1, W2p)


def _tc3_body(g2_ref, s2p_ref, d_ref, b2_ref, y_ref):
  d = d_ref[...]
  y = d[:, None] * (s2p_ref[0] + s2p_ref[1] + g2_ref[...]) + b2_ref[...][None, :]
  y_ref[...] = jnp.tanh(jnp.maximum(y, 0.0))[:, :D_OUT]


def _tc3(g2, s2p, d, b2p):
  return pl.pallas_call(
      _tc3_body,
      grid=(_GRID,),
      in_specs=[
          pl.BlockSpec((RB, D_OUT_PAD), lambda i: (i, 0)),
          pl.BlockSpec((NC, RB, D_OUT_PAD), lambda i: (0, i, 0)),
          pl.BlockSpec((RB,), lambda i: (i,)),
          pl.BlockSpec((D_OUT_PAD,), lambda i: (0,)),
      ],
      out_specs=pl.BlockSpec((RB, D_OUT), lambda i: (i, 0)),
      out_shape=jax.ShapeDtypeStruct((N_NODE, D_OUT), jnp.float32),
  )(g2, s2p, d, b2p)


def kernel(x, edge_index, W1, b1, W2, b2):
  ei = edge_index.astype(jnp.int32)
  W2p = jnp.pad(W2, ((0, 0), (0, D_OUT_PAD - W2.shape[1])))
  b2p = jnp.pad(b2, (0, D_OUT_PAD - b2.shape[0]))

  degp = _deg_kernel(ei)
  g, d = _tc1(x, W1, degp)
  sp = _scatter_kernel(g, ei, D_FEAT, ring=6, sidx_halves=1, dtype=jnp.bfloat16)
  g2 = _tc2(g, sp, d, b1, W2p)
  s2p = _scatter_kernel(g2, ei, D_OUT_PAD, ring=6, sidx_halves=1)
  y = _tc3(g2, s2p, d, b2p)
  return y
